# baseline (speedup 1.0000x reference)
import functools

import jax
import jax.numpy as jnp
from jax import lax
from jax.experimental import pallas as pl
from jax.experimental.pallas import tpu as pltpu
from jax.experimental.pallas import tpu_sc as plsc

D_MODEL = 64
SCALE = float(D_MODEL) ** 0.5
NUM_CORES = 2
NUM_SUBCORES = 16
NUM_WORKERS = NUM_CORES * NUM_SUBCORES
LANES = 16
BBLK = 128
FMT_BLK = 16384


def _tc_format_body(t_ref, o_ref):
    t = jnp.swapaxes(t_ref[...], 0, 1) * SCALE
    o_ref[:, :D_MODEL] = t[: FMT_BLK // 2]
    o_ref[:, D_MODEL:] = t[FMT_BLK // 2 :]


def _tc_format(tt):
    v = tt.shape[1]
    n_blk = (v + FMT_BLK - 1) // FMT_BLK
    return pl.pallas_call(
        _tc_format_body,
        grid=(n_blk,),
        in_specs=[pl.BlockSpec((D_MODEL, FMT_BLK), lambda i: (0, i))],
        out_specs=pl.BlockSpec((FMT_BLK // 2, 2 * D_MODEL), lambda i: (i, 0)),
        out_shape=jax.ShapeDtypeStruct((n_blk * FMT_BLK // 2, 2 * D_MODEL), jnp.float32),
    )(tt)


@functools.partial(jax.jit, static_argnums=(2, 3))
def _sc_embed(idx, table_scaled, batch, seq):
    n_rows = batch * seq
    n_units = n_rows // BBLK
    u_per_w = n_units // NUM_WORKERS
    nbb = batch // BBLK
    mesh = plsc.VectorSubcoreMesh(core_axis_name="c", subcore_axis_name="s")

    @functools.partial(
        pl.kernel,
        mesh=mesh,
        out_type=jax.ShapeDtypeStruct(
            (seq, D_MODEL // 8, nbb, 8, BBLK), jnp.float32
        ),
        scratch_types=[
            [pltpu.VMEM((BBLK,), jnp.int32) for _ in range(3)],
            [pltpu.VMEM((BBLK, D_MODEL), jnp.float32) for _ in range(2)],
            [pltpu.VMEM((D_MODEL // 8, 8, BBLK), jnp.float32) for _ in range(2)],
            [pltpu.SemaphoreType.DMA for _ in range(3)],
            [pltpu.SemaphoreType.DMA for _ in range(2)],
            [pltpu.SemaphoreType.DMA for _ in range(2)],
        ],
        compiler_params=pltpu.CompilerParams(
            use_tc_tiling_on_sc=False, needs_layout_passes=False
        ),
    )
    def k(idx_hbm, table_hbm, out_hbm, idxs, rows, stags, i_sems, g_sems, w_sems):
        wid = lax.axis_index("s") * NUM_CORES + lax.axis_index("c")
        base_u = wid * u_per_w

        def start_idx(u):
            b = u % 3
            return pltpu.async_copy(
                idx_hbm.at[pl.ds((base_u + u) * BBLK, BBLK)], idxs[b], i_sems[b]
            )

        def start_gather(u):
            return pltpu.async_copy(
                table_hbm.at[idxs[u % 3]], rows[u % 2], g_sems[u % 2]
            )

        def start_wb(u):
            ug = base_u + u
            s = ug // nbb
            tj = ug % nbb
            return pltpu.async_copy(
                stags[u % 2], out_hbm.at[s, :, tj], w_sems[u % 2]
            )

        row_base = [
            lax.broadcasted_iota(jnp.int32, (LANES,), 0) + kk * LANES
            for kk in range(BBLK // LANES)
        ]

        idx_copies = {u: start_idx(u) for u in range(min(2, u_per_w))}
        idx_copies.pop(0).wait()
        gathers = {0: start_gather(0)}
        writebacks = {}
        for u in range(u_per_w):
            gathers.pop(u).wait()
            if u + 1 < u_per_w:
                if u + 2 < u_per_w:
                    idx_copies[u + 2] = start_idx(u + 2)
                idx_copies.pop(u + 1).wait()
                gathers[u + 1] = start_gather(u + 1)
            if u >= 2:
                writebacks.pop(u - 2).wait()

            rbuf = rows[u % 2]
            sbuf = stags[u % 2]

            @plsc.parallel_loop(0, D_MODEL, step=1, unroll=2)
            def _transpose(d):
                col = jnp.broadcast_to(d, (LANES,)).astype(jnp.int32)
                ti = d // 8
                r = d % 8
                for kk in range(BBLK // LANES):
                    vec = plsc.load_gather(rbuf, [row_base[kk], col])
                    sbuf[ti, r, pl.ds(kk * LANES, LANES)] = vec

            writebacks[u] = start_wb(u)
        for u in sorted(writebacks):
            writebacks.pop(u).wait()

    return k(idx, table_scaled)


def kernel(x, table):
    b, s = x.shape
    n = b * s
    xf = x.T.reshape(n).astype(jnp.int32)
    half = FMT_BLK // 2
    blk = xf // FMT_BLK
    m = xf % FMT_BLK
    xg = blk * FMT_BLK + (m % half) * 2 + m // half
    tf = _tc_format(table.T)
    th = tf.reshape(tf.shape[0] * 2, D_MODEL)
    out5 = _sc_embed(xg, th, b, s)
    return out5.transpose(2, 4, 0, 1, 3).reshape(b, s, D_MODEL)


# speedup vs baseline: 1.3055x; 1.3055x over previous
import functools

import jax
import jax.numpy as jnp
from jax import lax
from jax.experimental import pallas as pl
from jax.experimental.pallas import tpu as pltpu
from jax.experimental.pallas import tpu_sc as plsc

D_MODEL = 64
D_PAD = 128
SCALE = float(D_MODEL) ** 0.5
NUM_CORES = 2
NUM_SUBCORES = 16
NUM_WORKERS = NUM_CORES * NUM_SUBCORES
CHUNK = 640
NBUF = 2
NIBUF = 3
FMT_BLK = 32768


def _tc_format_body(t_ref, o_ref):
    t = jnp.swapaxes(t_ref[...], 0, 1) * SCALE
    o_ref[:, :D_MODEL] = t[: FMT_BLK // 2]
    o_ref[:, D_MODEL:] = t[FMT_BLK // 2 :]


def _tc_format(tt):
    v = tt.shape[1]
    n_blk = (v + FMT_BLK - 1) // FMT_BLK
    return pl.pallas_call(
        _tc_format_body,
        grid=(n_blk,),
        in_specs=[pl.BlockSpec((D_MODEL, FMT_BLK), lambda i: (0, i))],
        out_specs=pl.BlockSpec((FMT_BLK // 2, 2 * D_MODEL), lambda i: (i, 0)),
        out_shape=jax.ShapeDtypeStruct((n_blk * FMT_BLK // 2, 2 * D_MODEL), jnp.float32),
    )(tt)


@functools.partial(jax.jit, static_argnums=(2,))
def _sc_embed(idx, table_scaled, n_rows):
    b_per_w = n_rows // NUM_WORKERS
    n_chunks = b_per_w // CHUNK
    mesh = plsc.VectorSubcoreMesh(core_axis_name="c", subcore_axis_name="s")

    @functools.partial(
        pl.kernel,
        mesh=mesh,
        out_type=jax.ShapeDtypeStruct((n_rows, D_PAD), jnp.float32),
        scratch_types=[
            [pltpu.VMEM((CHUNK,), jnp.int32) for _ in range(NIBUF)],
            [pltpu.VMEM((CHUNK, D_MODEL), jnp.float32) for _ in range(NBUF)],
            [pltpu.SemaphoreType.DMA for _ in range(NIBUF)],
            [pltpu.SemaphoreType.DMA for _ in range(NBUF)],
            [pltpu.SemaphoreType.DMA for _ in range(NBUF)],
        ],
        compiler_params=pltpu.CompilerParams(use_tc_tiling_on_sc=False),
    )
    def k(idx_hbm, table_hbm, out_hbm, idxs, rows, i_sems, g_sems, w_sems):
        wid = lax.axis_index("s") * NUM_CORES + lax.axis_index("c")
        base_w = wid * b_per_w

        def start_idx(c):
            b = c % NIBUF
            return pltpu.async_copy(
                idx_hbm.at[pl.ds(base_w + c * CHUNK, CHUNK)], idxs[b], i_sems[b]
            )

        def start_gather(c):
            return pltpu.async_copy(
                table_hbm.at[idxs[c % NIBUF]], rows[c % NBUF], g_sems[c % NBUF]
            )

        idx_copies = {c: start_idx(c) for c in range(min(2, n_chunks))}
        idx_copies.pop(0).wait()
        gathers = {0: start_gather(0)}
        writebacks = {}
        for c in range(n_chunks):
            gathers.pop(c).wait()
            if c + 1 < n_chunks:
                if c + 2 < n_chunks:
                    idx_copies[c + 2] = start_idx(c + 2)
                idx_copies.pop(c + 1).wait()
                if c + 1 >= NBUF:
                    writebacks.pop(c + 1 - NBUF).wait()
                gathers[c + 1] = start_gather(c + 1)

            writebacks[c] = pltpu.async_copy(
                rows[c % NBUF],
                out_hbm.at[pl.ds(base_w + c * CHUNK, CHUNK), pl.ds(0, D_MODEL)],
                w_sems[c % NBUF],
            )
        for c in sorted(writebacks):
            writebacks.pop(c).wait()

    return k(idx, table_scaled)


def kernel(x, table):
    b, s = x.shape
    n = b * s
    v = table.shape[0]
    xf = x.reshape(n).astype(jnp.int32)
    half = FMT_BLK // 2
    blk = xf // FMT_BLK
    m = xf % FMT_BLK
    xg = blk * FMT_BLK + (m % half) * 2 + m // half
    tf = _tc_format(table.T)
    th = tf.reshape(tf.shape[0] * 2, D_MODEL)
    out = _sc_embed(xg, th, n)
    return out.reshape(b, s, D_PAD)[:, :, :D_MODEL]
